# baseline (speedup 1.0000x reference)
import functools

import jax
import jax.numpy as jnp
from jax import lax
from jax.experimental import pallas as pl
from jax.experimental.pallas import tpu as pltpu
from jax.experimental.pallas import tpu_sc as plsc

BATCH = 16384
EMBED_DIM = 128
NUM_WORKERS = 32
PAIRS_PER_WORKER = BATCH // NUM_WORKERS
CHUNK = 128
NUM_CHUNKS = PAIRS_PER_WORKER // CHUNK
IDX_ROWS_PER_WORKER = PAIRS_PER_WORKER // CHUNK


def _sc_kernel(users_hbm, items_hbm, user_table, item_table, out_hbm,
               idx_u, idx_i, rows_u0, rows_u1, rows_i0, rows_i1, out_v,
               sem_u0, sem_u1, sem_i0, sem_i1):
    wid = lax.axis_index("s") * 2 + lax.axis_index("c")
    row0 = wid * IDX_ROWS_PER_WORKER

    pltpu.sync_copy(users_hbm.at[pl.ds(row0, IDX_ROWS_PER_WORKER)], idx_u)
    pltpu.sync_copy(items_hbm.at[pl.ds(row0, IDX_ROWS_PER_WORKER)], idx_i)

    lane = lax.iota(jnp.int32, 16)
    groups_per_chunk = CHUNK // 16
    D_STEP = 16

    bufs = [(rows_u0, rows_i0, sem_u0, sem_i0), (rows_u1, rows_i1, sem_u1, sem_i1)]

    def start(c):
        ru, ri, su, si = bufs[c % 2]
        cu = pltpu.async_copy(user_table.at[idx_u.at[c]], ru, su)
        ci = pltpu.async_copy(item_table.at[idx_i.at[c]], ri, si)
        return cu, ci

    inflight = start(0)
    for c in range(NUM_CHUNKS):
        cu, ci = inflight
        cu.wait()
        ci.wait()
        if c + 1 < NUM_CHUNKS:
            inflight = start(c + 1)
        ru, ri = bufs[c % 2][0], bufs[c % 2][1]

        @plsc.parallel_loop(0, groups_per_chunk, unroll=1)
        def group_body(g, ru=ru, ri=ri, c=c):
            rowvec = g * 16 + lane
            zero = jnp.zeros((16,), jnp.float32)
            init = (jnp.zeros((16,), jnp.int32), zero, zero, zero, zero)

            @plsc.parallel_loop(0, EMBED_DIM, step=D_STEP, carry=init)
            def d_loop(dbase, carry, ru=ru, ri=ri, rowvec=rowvec):
                col, a0, a1, a2, a3 = carry
                accs = [a0, a1, a2, a3]
                for t in range(D_STEP):
                    u = plsc.load_gather(ru, [rowvec, col])
                    v = plsc.load_gather(ri, [rowvec, col])
                    accs[t % 4] = accs[t % 4] + u * v
                    col = col + 1
                return (col, accs[0], accs[1], accs[2], accs[3])

            _, a0, a1, a2, a3 = d_loop
            res = (a0 + a1) + (a2 + a3)
            out_v[c * groups_per_chunk + g] = 1.0 / (1.0 + jnp.exp(-res))

    pltpu.sync_copy(
        out_v, out_hbm.at[pl.ds(wid * (PAIRS_PER_WORKER // 16), PAIRS_PER_WORKER // 16)])


@functools.partial(jax.jit, static_argnums=())
def _run(users2d, items2d, user_table, item_table):
    mesh = plsc.VectorSubcoreMesh(core_axis_name="c", subcore_axis_name="s")
    f = pl.kernel(
        _sc_kernel,
        mesh=mesh,
        compiler_params=pltpu.CompilerParams(needs_layout_passes=False),
        out_type=jax.ShapeDtypeStruct((BATCH // 16, 16), jnp.float32),
        scratch_types=[
            pltpu.VMEM((IDX_ROWS_PER_WORKER, CHUNK), jnp.int32),
            pltpu.VMEM((IDX_ROWS_PER_WORKER, CHUNK), jnp.int32),
            pltpu.VMEM((CHUNK, EMBED_DIM), jnp.float32),
            pltpu.VMEM((CHUNK, EMBED_DIM), jnp.float32),
            pltpu.VMEM((CHUNK, EMBED_DIM), jnp.float32),
            pltpu.VMEM((CHUNK, EMBED_DIM), jnp.float32),
            pltpu.VMEM((PAIRS_PER_WORKER // 16, 16), jnp.float32),
            pltpu.SemaphoreType.DMA,
            pltpu.SemaphoreType.DMA,
            pltpu.SemaphoreType.DMA,
            pltpu.SemaphoreType.DMA,
        ],
    )
    return f(users2d, items2d, user_table, item_table)


def kernel(data, user_table, item_table):
    users2d = data[:, 0].reshape(BATCH // CHUNK, CHUNK)
    items2d = data[:, 1].reshape(BATCH // CHUNK, CHUNK)
    return _run(users2d, items2d, user_table, item_table).reshape(BATCH)


# speedup vs baseline: 2.7076x; 2.7076x over previous
import functools

import jax
import jax.numpy as jnp
from jax import lax
from jax.experimental import pallas as pl
from jax.experimental.pallas import tpu as pltpu
from jax.experimental.pallas import tpu_sc as plsc

BATCH = 16384
EMBED_DIM = 128
NUM_WORKERS = 32
PAIRS_PER_WORKER = BATCH // NUM_WORKERS
CHUNK = 128
NUM_CHUNKS = PAIRS_PER_WORKER // CHUNK
IDX_ROWS_PER_WORKER = PAIRS_PER_WORKER // CHUNK


def _sc_kernel(users_hbm, items_hbm, user_table, item_table, out_hbm,
               idx_u, idx_i, rows_u0, rows_u1, rows_i0, rows_i1, out_v,
               sem_u0, sem_u1, sem_i0, sem_i1):
    wid = lax.axis_index("s") * 2 + lax.axis_index("c")
    row0 = wid * IDX_ROWS_PER_WORKER

    pltpu.sync_copy(users_hbm.at[pl.ds(row0, IDX_ROWS_PER_WORKER)], idx_u)
    pltpu.sync_copy(items_hbm.at[pl.ds(row0, IDX_ROWS_PER_WORKER)], idx_i)

    lane = lax.iota(jnp.int32, 16)
    groups_per_chunk = CHUNK // 16
    diags = [jnp.bitwise_and(lane + j, 15) for j in range(16)]

    bufs = [(rows_u0, rows_i0, sem_u0, sem_i0), (rows_u1, rows_i1, sem_u1, sem_i1)]

    def start(c):
        ru, ri, su, si = bufs[c % 2]
        cu = pltpu.async_copy(user_table.at[idx_u.at[c]], ru, su)
        ci = pltpu.async_copy(item_table.at[idx_i.at[c]], ri, si)
        return cu, ci

    inflight = start(0)
    for c in range(NUM_CHUNKS):
        cu, ci = inflight
        cu.wait()
        ci.wait()
        if c + 1 < NUM_CHUNKS:
            inflight = start(c + 1)
        ru, ri = bufs[c % 2][0], bufs[c % 2][1]

        @plsc.parallel_loop(0, groups_per_chunk, unroll=1)
        def group_body(g, ru=ru, ri=ri, c=c):
            rowvec = g * 16 + lane
            zero = jnp.zeros((16,), jnp.float32)
            init = (zero, zero)

            @plsc.parallel_loop(0, EMBED_DIM // 16, carry=init)
            def m_loop(m, carry, ru=ru, ri=ri, rowvec=rowvec):
                a0, a1 = carry
                mvec = jnp.full((16,), m * 16, jnp.int32)
                for j in range(16):
                    col = diags[j] + mvec
                    u = plsc.load_gather(ru, [rowvec, col])
                    v = plsc.load_gather(ri, [rowvec, col])
                    if j % 2 == 0:
                        a0 = a0 + u * v
                    else:
                        a1 = a1 + u * v
                return (a0, a1)

            a0, a1 = m_loop
            res = a0 + a1
            out_v[c * groups_per_chunk + g] = 1.0 / (1.0 + jnp.exp(-res))

    pltpu.sync_copy(
        out_v, out_hbm.at[pl.ds(wid * (PAIRS_PER_WORKER // 16), PAIRS_PER_WORKER // 16)])


@functools.partial(jax.jit, static_argnums=())
def _run(users2d, items2d, user_table, item_table):
    mesh = plsc.VectorSubcoreMesh(core_axis_name="c", subcore_axis_name="s")
    f = pl.kernel(
        _sc_kernel,
        mesh=mesh,
        compiler_params=pltpu.CompilerParams(needs_layout_passes=False),
        out_type=jax.ShapeDtypeStruct((BATCH // 16, 16), jnp.float32),
        scratch_types=[
            pltpu.VMEM((IDX_ROWS_PER_WORKER, CHUNK), jnp.int32),
            pltpu.VMEM((IDX_ROWS_PER_WORKER, CHUNK), jnp.int32),
            pltpu.VMEM((CHUNK, EMBED_DIM), jnp.float32),
            pltpu.VMEM((CHUNK, EMBED_DIM), jnp.float32),
            pltpu.VMEM((CHUNK, EMBED_DIM), jnp.float32),
            pltpu.VMEM((CHUNK, EMBED_DIM), jnp.float32),
            pltpu.VMEM((PAIRS_PER_WORKER // 16, 16), jnp.float32),
            pltpu.SemaphoreType.DMA,
            pltpu.SemaphoreType.DMA,
            pltpu.SemaphoreType.DMA,
            pltpu.SemaphoreType.DMA,
        ],
    )
    return f(users2d, items2d, user_table, item_table)


def kernel(data, user_table, item_table):
    users2d = data[:, 0].reshape(BATCH // CHUNK, CHUNK)
    items2d = data[:, 1].reshape(BATCH // CHUNK, CHUNK)
    return _run(users2d, items2d, user_table, item_table).reshape(BATCH)


# speedup vs baseline: 2.8269x; 1.0441x over previous
import functools

import jax
import jax.numpy as jnp
from jax import lax
from jax.experimental import pallas as pl
from jax.experimental.pallas import tpu as pltpu
from jax.experimental.pallas import tpu_sc as plsc

BATCH = 16384
EMBED_DIM = 128
NUM_WORKERS = 32
PAIRS_PER_WORKER = BATCH // NUM_WORKERS
CHUNK = 128
NUM_CHUNKS = PAIRS_PER_WORKER // CHUNK
IDX_ROWS_PER_WORKER = PAIRS_PER_WORKER // CHUNK


def _sc_kernel(users_hbm, items_hbm, user_table, item_table, out_hbm,
               idx_u, idx_i, rows_u0, rows_u1, rows_i0, rows_i1, out_v,
               sem_u0, sem_u1, sem_i0, sem_i1):
    wid = lax.axis_index("s") * 2 + lax.axis_index("c")
    row0 = wid * IDX_ROWS_PER_WORKER

    pltpu.sync_copy(users_hbm.at[pl.ds(row0, IDX_ROWS_PER_WORKER)], idx_u)
    pltpu.sync_copy(items_hbm.at[pl.ds(row0, IDX_ROWS_PER_WORKER)], idx_i)

    lane = lax.iota(jnp.int32, 16)
    groups_per_chunk = CHUNK // 16
    diags = [jnp.bitwise_and(lane + j, 15) for j in range(16)]

    bufs = [(rows_u0, rows_i0, sem_u0, sem_i0), (rows_u1, rows_i1, sem_u1, sem_i1)]

    def start(c):
        ru, ri, su, si = bufs[c % 2]
        cu = pltpu.async_copy(user_table.at[idx_u.at[c]], ru, su)
        ci = pltpu.async_copy(item_table.at[idx_i.at[c]], ri, si)
        return cu, ci

    inflight = start(0)
    for c in range(NUM_CHUNKS):
        cu, ci = inflight
        cu.wait()
        ci.wait()
        if c + 1 < NUM_CHUNKS:
            inflight = start(c + 1)
        ru, ri = bufs[c % 2][0], bufs[c % 2][1]

        @plsc.parallel_loop(0, groups_per_chunk, unroll=1)
        def group_body(g, ru=ru, ri=ri, c=c):
            rowvec = g * 16 + lane
            zero = jnp.zeros((16,), jnp.float32)
            init = (zero, zero)

            @plsc.parallel_loop(0, EMBED_DIM // 16, carry=init, unroll=2)
            def m_loop(m, carry, ru=ru, ri=ri, rowvec=rowvec):
                a0, a1 = carry
                mvec = jnp.full((16,), m * 16, jnp.int32)
                for j in range(16):
                    col = diags[j] + mvec
                    u = plsc.load_gather(ru, [rowvec, col])
                    v = plsc.load_gather(ri, [rowvec, col])
                    if j % 2 == 0:
                        a0 = a0 + u * v
                    else:
                        a1 = a1 + u * v
                return (a0, a1)

            a0, a1 = m_loop
            res = a0 + a1
            out_v[pl.ds((c * groups_per_chunk + g) * 16, 16)] = (
                1.0 / (1.0 + jnp.exp(-res)))

    pltpu.sync_copy(
        out_v, out_hbm.at[pl.ds(wid * PAIRS_PER_WORKER, PAIRS_PER_WORKER)])


@functools.partial(jax.jit, static_argnums=())
def _run(users2d, items2d, user_table, item_table):
    mesh = plsc.VectorSubcoreMesh(core_axis_name="c", subcore_axis_name="s")
    f = pl.kernel(
        _sc_kernel,
        mesh=mesh,
        compiler_params=pltpu.CompilerParams(needs_layout_passes=False),
        out_type=jax.ShapeDtypeStruct((BATCH,), jnp.float32),
        scratch_types=[
            pltpu.VMEM((IDX_ROWS_PER_WORKER, CHUNK), jnp.int32),
            pltpu.VMEM((IDX_ROWS_PER_WORKER, CHUNK), jnp.int32),
            pltpu.VMEM((CHUNK, EMBED_DIM), jnp.float32),
            pltpu.VMEM((CHUNK, EMBED_DIM), jnp.float32),
            pltpu.VMEM((CHUNK, EMBED_DIM), jnp.float32),
            pltpu.VMEM((CHUNK, EMBED_DIM), jnp.float32),
            pltpu.VMEM((PAIRS_PER_WORKER,), jnp.float32),
            pltpu.SemaphoreType.DMA,
            pltpu.SemaphoreType.DMA,
            pltpu.SemaphoreType.DMA,
            pltpu.SemaphoreType.DMA,
        ],
    )
    return f(users2d, items2d, user_table, item_table)


def kernel(data, user_table, item_table):
    users2d = data[:, 0].reshape(BATCH // CHUNK, CHUNK)
    items2d = data[:, 1].reshape(BATCH // CHUNK, CHUNK)
    return _run(users2d, items2d, user_table, item_table)


# speedup vs baseline: 2.8273x; 1.0001x over previous
import functools

import jax
import jax.numpy as jnp
from jax import lax
from jax.experimental import pallas as pl
from jax.experimental.pallas import tpu as pltpu
from jax.experimental.pallas import tpu_sc as plsc

BATCH = 16384
EMBED_DIM = 128
NUM_WORKERS = 32
PAIRS_PER_WORKER = BATCH // NUM_WORKERS
CHUNK = 128
NUM_CHUNKS = PAIRS_PER_WORKER // CHUNK
IDX_ROWS_PER_WORKER = PAIRS_PER_WORKER // CHUNK


def _sc_kernel(users_hbm, items_hbm, user_table, item_table, out_hbm,
               idx_u, idx_i, rows_u0, rows_u1, rows_i0, rows_i1, out_v,
               sem_u0, sem_u1, sem_i0, sem_i1):
    wid = lax.axis_index("s") * 2 + lax.axis_index("c")
    base = wid * PAIRS_PER_WORKER

    pltpu.sync_copy(users_hbm.at[pl.ds(base, PAIRS_PER_WORKER)], idx_u)
    pltpu.sync_copy(items_hbm.at[pl.ds(base, PAIRS_PER_WORKER)], idx_i)

    lane = lax.iota(jnp.int32, 16)
    groups_per_chunk = CHUNK // 16
    diags = [jnp.bitwise_and(lane + j, 15) for j in range(16)]

    bufs = [(rows_u0, rows_i0, sem_u0, sem_i0), (rows_u1, rows_i1, sem_u1, sem_i1)]

    def start(c):
        ru, ri, su, si = bufs[c % 2]
        cu = pltpu.async_copy(
            user_table.at[idx_u.at[pl.ds(c * CHUNK, CHUNK)]], ru, su)
        ci = pltpu.async_copy(
            item_table.at[idx_i.at[pl.ds(c * CHUNK, CHUNK)]], ri, si)
        return cu, ci

    inflight = start(0)
    for c in range(NUM_CHUNKS):
        cu, ci = inflight
        cu.wait()
        ci.wait()
        if c + 1 < NUM_CHUNKS:
            inflight = start(c + 1)
        ru, ri = bufs[c % 2][0], bufs[c % 2][1]

        @plsc.parallel_loop(0, groups_per_chunk, unroll=1)
        def group_body(g, ru=ru, ri=ri, c=c):
            rowvec = g * 16 + lane
            zero = jnp.zeros((16,), jnp.float32)
            init = (zero, zero)

            @plsc.parallel_loop(0, EMBED_DIM // 16, carry=init, unroll=2)
            def m_loop(m, carry, ru=ru, ri=ri, rowvec=rowvec):
                a0, a1 = carry
                mvec = jnp.full((16,), m * 16, jnp.int32)
                for j in range(16):
                    col = diags[j] + mvec
                    u = plsc.load_gather(ru, [rowvec, col])
                    v = plsc.load_gather(ri, [rowvec, col])
                    if j % 2 == 0:
                        a0 = a0 + u * v
                    else:
                        a1 = a1 + u * v
                return (a0, a1)

            a0, a1 = m_loop
            res = a0 + a1
            out_v[pl.ds((c * groups_per_chunk + g) * 16, 16)] = (
                1.0 / (1.0 + jnp.exp(-res)))

    pltpu.sync_copy(
        out_v, out_hbm.at[pl.ds(wid * PAIRS_PER_WORKER, PAIRS_PER_WORKER)])


@functools.partial(jax.jit, static_argnums=())
def _run(users, items, user_table, item_table):
    mesh = plsc.VectorSubcoreMesh(core_axis_name="c", subcore_axis_name="s")
    f = pl.kernel(
        _sc_kernel,
        mesh=mesh,
        compiler_params=pltpu.CompilerParams(needs_layout_passes=False),
        out_type=jax.ShapeDtypeStruct((BATCH,), jnp.float32),
        scratch_types=[
            pltpu.VMEM((PAIRS_PER_WORKER,), jnp.int32),
            pltpu.VMEM((PAIRS_PER_WORKER,), jnp.int32),
            pltpu.VMEM((CHUNK, EMBED_DIM), jnp.float32),
            pltpu.VMEM((CHUNK, EMBED_DIM), jnp.float32),
            pltpu.VMEM((CHUNK, EMBED_DIM), jnp.float32),
            pltpu.VMEM((CHUNK, EMBED_DIM), jnp.float32),
            pltpu.VMEM((PAIRS_PER_WORKER,), jnp.float32),
            pltpu.SemaphoreType.DMA,
            pltpu.SemaphoreType.DMA,
            pltpu.SemaphoreType.DMA,
            pltpu.SemaphoreType.DMA,
        ],
    )
    return f(users, items, user_table, item_table)


def kernel(data, user_table, item_table):
    return _run(data[:, 0], data[:, 1], user_table, item_table)


# speedup vs baseline: 2.8650x; 1.0134x over previous
import functools

import jax
import jax.numpy as jnp
from jax import lax
from jax.experimental import pallas as pl
from jax.experimental.pallas import tpu as pltpu
from jax.experimental.pallas import tpu_sc as plsc

BATCH = 16384
EMBED_DIM = 128
NUM_WORKERS = 32
PAIRS_PER_WORKER = BATCH // NUM_WORKERS
CHUNK = 128
NUM_CHUNKS = PAIRS_PER_WORKER // CHUNK
IDX_ROWS_PER_WORKER = PAIRS_PER_WORKER // CHUNK


def _sc_kernel(users_hbm, items_hbm, user_table, item_table, out_hbm,
               idx_u, idx_i, rows_u0, rows_u1, rows_i0, rows_i1, out_v,
               sem_u0, sem_u1, sem_i0, sem_i1):
    wid = lax.axis_index("s") * 2 + lax.axis_index("c")
    base = wid * PAIRS_PER_WORKER

    pltpu.sync_copy(users_hbm.at[pl.ds(base, PAIRS_PER_WORKER)], idx_u)
    pltpu.sync_copy(items_hbm.at[pl.ds(base, PAIRS_PER_WORKER)], idx_i)

    lane = lax.iota(jnp.int32, 16)
    groups_per_chunk = CHUNK // 16
    diags = [jnp.bitwise_and(lane + j, 15) for j in range(16)]

    bufs = [(rows_u0, rows_i0, sem_u0, sem_i0), (rows_u1, rows_i1, sem_u1, sem_i1)]

    def start(c, b):
        ru, ri, su, si = bufs[b]
        cu = pltpu.async_copy(
            user_table.at[idx_u.at[pl.ds(c * CHUNK, CHUNK)]], ru, su)
        ci = pltpu.async_copy(
            item_table.at[idx_i.at[pl.ds(c * CHUNK, CHUNK)]], ri, si)
        return cu, ci

    start(0, 0)
    start(1, 1)

    def compute_chunk(c, b):
        ru, ri, su, si = bufs[b]
        pltpu.make_async_copy(
            user_table.at[idx_u.at[pl.ds(0, CHUNK)]], ru, su).wait()
        pltpu.make_async_copy(
            item_table.at[idx_i.at[pl.ds(0, CHUNK)]], ri, si).wait()

        @plsc.parallel_loop(0, groups_per_chunk, unroll=1)
        def group_body(g):
            rowvec = g * 16 + lane
            zero = jnp.zeros((16,), jnp.float32)
            init = (zero, zero)

            @plsc.parallel_loop(0, EMBED_DIM // 16, carry=init, unroll=2)
            def m_loop(m, carry):
                a0, a1 = carry
                mvec = jnp.full((16,), m * 16, jnp.int32)
                for j in range(16):
                    col = diags[j] + mvec
                    u = plsc.load_gather(ru, [rowvec, col])
                    v = plsc.load_gather(ri, [rowvec, col])
                    if j % 2 == 0:
                        a0 = a0 + u * v
                    else:
                        a1 = a1 + u * v
                return (a0, a1)

            a0, a1 = m_loop
            res = a0 + a1
            out_v[pl.ds((c * groups_per_chunk + g) * 16, 16)] = (
                1.0 / (1.0 + jnp.exp(-res)))

        @pl.when(c + 2 < NUM_CHUNKS)
        def _():
            start(c + 2, b)

    def outer_body(o, carry):
        compute_chunk(o * 2, 0)
        compute_chunk(o * 2 + 1, 1)
        return carry

    lax.fori_loop(0, NUM_CHUNKS // 2, outer_body, 0)

    pltpu.sync_copy(
        out_v, out_hbm.at[pl.ds(wid * PAIRS_PER_WORKER, PAIRS_PER_WORKER)])


@functools.partial(jax.jit, static_argnums=())
def _run(users, items, user_table, item_table):
    mesh = plsc.VectorSubcoreMesh(core_axis_name="c", subcore_axis_name="s")
    f = pl.kernel(
        _sc_kernel,
        mesh=mesh,
        compiler_params=pltpu.CompilerParams(needs_layout_passes=False),
        out_type=jax.ShapeDtypeStruct((BATCH,), jnp.float32),
        scratch_types=[
            pltpu.VMEM((PAIRS_PER_WORKER,), jnp.int32),
            pltpu.VMEM((PAIRS_PER_WORKER,), jnp.int32),
            pltpu.VMEM((CHUNK, EMBED_DIM), jnp.float32),
            pltpu.VMEM((CHUNK, EMBED_DIM), jnp.float32),
            pltpu.VMEM((CHUNK, EMBED_DIM), jnp.float32),
            pltpu.VMEM((CHUNK, EMBED_DIM), jnp.float32),
            pltpu.VMEM((PAIRS_PER_WORKER,), jnp.float32),
            pltpu.SemaphoreType.DMA,
            pltpu.SemaphoreType.DMA,
            pltpu.SemaphoreType.DMA,
            pltpu.SemaphoreType.DMA,
        ],
    )
    return f(users, items, user_table, item_table)


def kernel(data, user_table, item_table):
    return _run(data[:, 0], data[:, 1], user_table, item_table)
